# Initial kernel scaffold; baseline (speedup 1.0000x reference)
import jax
import jax.numpy as jnp
from jax.experimental import pallas as pl


def kernel(user_indices, item_indices, seq, mask, emb_item, linear_w, linear_b, self_linear_w, self_linear_b):
    raise NotImplementedError("write your pallas kernel here")


# speedup vs baseline: 9.7479x; 9.7479x over previous
import jax
import jax.numpy as jnp
from jax import lax
from jax.experimental import pallas as pl
from jax.experimental.pallas import tpu as pltpu
from jax.experimental.pallas import tpu_sc as plsc

B, L, K, V, D = 4096, 200, 20, 1000000, 32

NC, NS = 2, 16
NW = NC * NS

PACK = 128 // D
VP = V // PACK


_TOPK_ROWS = 256


def _topk_body(seq_ref, mask_ref, g_ref, out_ref):
  seq = seq_ref[...]
  mask = mask_ref[...]
  g = g_ref[...]
  neg_inf = jnp.float32(-jnp.inf)

  n_valid = jnp.sum(mask, axis=1, keepdims=True)
  w = jnp.where(n_valid > K, seq * mask, seq).astype(jnp.float32)
  logw = jnp.where(w > 0, jnp.log(jnp.maximum(w, 1e-30)), neg_inf)
  scores = jnp.where(logw > neg_inf, logw + g, neg_inf)

  col = lax.broadcasted_iota(jnp.int32, scores.shape, 1)
  big = jnp.int32(1 << 30)
  taken = jnp.zeros(scores.shape, jnp.bool_)
  for k in range(K):
    m = jnp.max(scores, axis=1, keepdims=True)
    is_max = (scores == m) & (~taken)
    idx = jnp.min(jnp.where(is_max, col, big), axis=1, keepdims=True)
    sel = col == idx
    val = jnp.sum(jnp.where(sel, seq, 0), axis=1, keepdims=True)
    out_ref[:, k : k + 1] = val
    taken = taken | sel
    scores = jnp.where(sel, neg_inf, scores)


def _run_topk(seq, mask, g):
  grid = B // _TOPK_ROWS
  return pl.pallas_call(
      _topk_body,
      grid=(grid,),
      in_specs=[
          pl.BlockSpec((_TOPK_ROWS, L), lambda i: (i, 0)),
          pl.BlockSpec((_TOPK_ROWS, L), lambda i: (i, 0)),
          pl.BlockSpec((_TOPK_ROWS, L), lambda i: (i, 0)),
      ],
      out_specs=pl.BlockSpec((_TOPK_ROWS, K), lambda i: (i, 0)),
      out_shape=jax.ShapeDtypeStruct((B, K), jnp.int32),
  )(seq, mask, g)


SEQ_N = B * L
SAMP_N = B * K
GCHUNK = 256
SEQ_PW = SEQ_N // NW
SAMP_PW = SAMP_N // NW
SEQ_CH = SEQ_PW // GCHUNK
SAMP_CH = SAMP_PW // GCHUNK


def _sc_gather_body(table, seq_idx, samp_idx, seq_out, samp_out,
                    idx_v, sidx_v, rows_v, sem):
  wid = lax.axis_index("s") * NC + lax.axis_index("c")

  seq_base = wid * SEQ_PW
  samp_base = wid * SAMP_PW

  pltpu.sync_copy(seq_idx.at[pl.ds(seq_base, SEQ_PW)], idx_v)
  pltpu.sync_copy(samp_idx.at[pl.ds(samp_base, SAMP_PW)], sidx_v)

  def seq_step(j, _):
    pltpu.async_copy(
        table.at[idx_v.at[pl.ds(j * GCHUNK, GCHUNK)]], rows_v, sem).wait()
    pltpu.sync_copy(rows_v, seq_out.at[pl.ds(seq_base + j * GCHUNK, GCHUNK)])
    return 0

  lax.fori_loop(0, SEQ_CH, seq_step, 0)

  def samp_step(j, _):
    pltpu.async_copy(
        table.at[sidx_v.at[pl.ds(j * GCHUNK, GCHUNK)]], rows_v, sem).wait()
    pltpu.sync_copy(rows_v, samp_out.at[pl.ds(samp_base + j * GCHUNK, GCHUNK)])
    return 0

  lax.fori_loop(0, SAMP_CH, samp_step, 0)


def _run_sc_gather(table_packed, seq_pidx, samp_pidx):
  mesh = plsc.VectorSubcoreMesh(
      core_axis_name="c", subcore_axis_name="s", num_cores=NC,
      num_subcores=NS)
  f = pl.kernel(
      _sc_gather_body,
      out_type=[
          jax.ShapeDtypeStruct((SEQ_N, 128), jnp.float32),
          jax.ShapeDtypeStruct((SAMP_N, 128), jnp.float32),
      ],
      mesh=mesh,
      scratch_types=[
          pltpu.VMEM((SEQ_PW,), jnp.int32),
          pltpu.VMEM((SAMP_PW,), jnp.int32),
          pltpu.VMEM((GCHUNK, 128), jnp.float32),
          pltpu.SemaphoreType.DMA,
      ],
  )
  return f(table_packed, seq_pidx, samp_pidx)


_BB = 16
_XRB = _BB * L
_NRB = _BB * K


def _extract32(x128, grp):
  out = jnp.zeros((x128.shape[0], D), jnp.float32)
  for k in range(PACK):
    piece = x128[:, k * D : (k + 1) * D]
    out = out + jnp.where(grp == k, piece, 0.0)
  return out


def _dense_body(xs_ref, g_ref, ns_ref, sg_ref, wself_ref, wn_ref, bs_ref,
                bl_ref, out_ref):
  ns32 = _extract32(ns_ref[...], sg_ref[...])
  rq = lax.broadcasted_iota(jnp.int32, (_BB, _NRB), 1) // K
  jq = lax.broadcasted_iota(jnp.int32, (_BB, _NRB), 0)
  q = (rq == jq).astype(jnp.float32)
  srow = jnp.dot(q, ns32, preferred_element_type=jnp.float32)
  rowb = jnp.dot(srow, wn_ref[...], preferred_element_type=jnp.float32)
  rowb = rowb + bl_ref[...]

  rp = lax.broadcasted_iota(jnp.int32, (_XRB, _BB), 0) // L
  jp = lax.broadcasted_iota(jnp.int32, (_XRB, _BB), 1)
  p = (rp == jp).astype(jnp.float32)
  rowb_exp = jnp.dot(p, rowb, preferred_element_type=jnp.float32)

  xs32 = _extract32(xs_ref[...], g_ref[...])
  y = jnp.dot(xs32, wself_ref[...], preferred_element_type=jnp.float32)
  y = y + bs_ref[...] + rowb_exp
  out_ref[...] = jnp.maximum(y, 0.0)


def _run_dense(xs, g, ns, sg, wself, wn, bs, bl):
  grid = B // _BB
  return pl.pallas_call(
      _dense_body,
      grid=(grid,),
      in_specs=[
          pl.BlockSpec((_XRB, 128), lambda i: (i, 0)),
          pl.BlockSpec((_XRB, 1), lambda i: (i, 0)),
          pl.BlockSpec((_NRB, 128), lambda i: (i, 0)),
          pl.BlockSpec((_NRB, 1), lambda i: (i, 0)),
          pl.BlockSpec((D, D), lambda i: (0, 0)),
          pl.BlockSpec((D, D), lambda i: (0, 0)),
          pl.BlockSpec((1, D), lambda i: (0, 0)),
          pl.BlockSpec((1, D), lambda i: (0, 0)),
      ],
      out_specs=pl.BlockSpec((_XRB, D), lambda i: (i, 0)),
      out_shape=jax.ShapeDtypeStruct((SEQ_N, D), jnp.float32),
  )(xs, g, ns, sg, wself, wn, bs, bl)


@jax.jit
def _kernel_impl(seq, mask, emb_item, linear_w, linear_b, self_linear_w,
                 self_linear_b):
  gum = jax.random.gumbel(jax.random.key(42), (B, L), dtype=jnp.float32)

  sampled = _run_topk(seq, mask, gum)

  table_packed = emb_item.reshape(VP, 128)
  seq_flat = seq.reshape(SEQ_N)
  samp_flat = sampled.reshape(SAMP_N)

  seq_rows, samp_rows = _run_sc_gather(
      table_packed, seq_flat >> 2, samp_flat >> 2)

  out = _run_dense(
      seq_rows, (seq_flat & 3).reshape(SEQ_N, 1),
      samp_rows, (samp_flat & 3).reshape(SAMP_N, 1),
      self_linear_w.T, linear_w.T / K,
      self_linear_b.reshape(1, D), linear_b.reshape(1, D))
  return out.reshape(B, L, D)


def kernel(user_indices, item_indices, seq, mask, emb_item, linear_w,
           linear_b, self_linear_w, self_linear_b):
  del user_indices, item_indices
  return _kernel_impl(seq, mask, emb_item, linear_w, linear_b,
                      self_linear_w, self_linear_b)


# speedup vs baseline: 9.7895x; 1.0043x over previous
import jax
import jax.numpy as jnp
from jax import lax
from jax.experimental import pallas as pl
from jax.experimental.pallas import tpu as pltpu
from jax.experimental.pallas import tpu_sc as plsc

B, L, K, V, D = 4096, 200, 20, 1000000, 32

NC, NS = 2, 16
NW = NC * NS

PACK = 128 // D
VP = V // PACK


_TOPK_ROWS = 256


def _topk_body(seq_ref, mask_ref, g_ref, out_ref):
  seq = seq_ref[...]
  mask = mask_ref[...]
  g = g_ref[...]
  neg_inf = jnp.float32(-jnp.inf)

  n_valid = jnp.sum(mask, axis=1, keepdims=True)
  w = jnp.where(n_valid > K, seq * mask, seq).astype(jnp.float32)
  logw = jnp.where(w > 0, jnp.log(jnp.maximum(w, 1e-30)), neg_inf)
  scores = jnp.where(logw > neg_inf, logw + g, neg_inf)

  col = lax.broadcasted_iota(jnp.int32, scores.shape, 1)
  big = jnp.int32(1 << 30)
  taken = jnp.zeros(scores.shape, jnp.bool_)
  for k in range(K):
    m = jnp.max(scores, axis=1, keepdims=True)
    is_max = (scores == m) & (~taken)
    idx = jnp.min(jnp.where(is_max, col, big), axis=1, keepdims=True)
    sel = col == idx
    val = jnp.sum(jnp.where(sel, seq, 0), axis=1, keepdims=True)
    out_ref[:, k : k + 1] = val
    taken = taken | sel
    scores = jnp.where(sel, neg_inf, scores)


def _run_topk(seq, mask, g):
  grid = B // _TOPK_ROWS
  return pl.pallas_call(
      _topk_body,
      grid=(grid,),
      in_specs=[
          pl.BlockSpec((_TOPK_ROWS, L), lambda i: (i, 0)),
          pl.BlockSpec((_TOPK_ROWS, L), lambda i: (i, 0)),
          pl.BlockSpec((_TOPK_ROWS, L), lambda i: (i, 0)),
      ],
      out_specs=pl.BlockSpec((_TOPK_ROWS, K), lambda i: (i, 0)),
      out_shape=jax.ShapeDtypeStruct((B, K), jnp.int32),
  )(seq, mask, g)


SEQ_N = B * L
SAMP_N = B * K
GCHUNK = 256
SEQ_PW = SEQ_N // NW
SAMP_PW = SAMP_N // NW
SEQ_CH = SEQ_PW // GCHUNK
SAMP_CH = SAMP_PW // GCHUNK


def _sc_gather_body(table, seq_idx, samp_idx, seq_out, samp_out,
                    idx_v, sidx_v, rows_a, rows_b, sem_a, sem_b):
  wid = lax.axis_index("s") * NC + lax.axis_index("c")

  seq_base = wid * SEQ_PW
  samp_base = wid * SAMP_PW

  pltpu.sync_copy(seq_idx.at[pl.ds(seq_base, SEQ_PW)], idx_v)
  pltpu.sync_copy(samp_idx.at[pl.ds(samp_base, SAMP_PW)], sidx_v)

  def gather_all(idx, out, base, n_chunks):
    def start(j, buf, sem):
      pltpu.async_copy(
          table.at[idx.at[pl.ds(j * GCHUNK, GCHUNK)]], buf, sem)

    def drain(j, buf, sem):
      pltpu.make_async_copy(
          table.at[idx.at[pl.ds(j * GCHUNK, GCHUNK)]], buf, sem).wait()
      pltpu.sync_copy(buf, out.at[pl.ds(base + j * GCHUNK, GCHUNK)])

    start(0, rows_a, sem_a)

    def pair(t, _):
      j = t * 2
      start(j + 1, rows_b, sem_b)
      drain(j, rows_a, sem_a)
      start(j + 2, rows_a, sem_a)
      drain(j + 1, rows_b, sem_b)
      return 0

    lax.fori_loop(0, n_chunks // 2 - 1, pair, 0)
    j = n_chunks - 2
    start(j + 1, rows_b, sem_b)
    drain(j, rows_a, sem_a)
    drain(j + 1, rows_b, sem_b)

  gather_all(idx_v, seq_out, seq_base, SEQ_CH)
  gather_all(sidx_v, samp_out, samp_base, SAMP_CH)


def _run_sc_gather(table_packed, seq_pidx, samp_pidx):
  mesh = plsc.VectorSubcoreMesh(
      core_axis_name="c", subcore_axis_name="s", num_cores=NC,
      num_subcores=NS)
  f = pl.kernel(
      _sc_gather_body,
      out_type=[
          jax.ShapeDtypeStruct((SEQ_N, 128), jnp.float32),
          jax.ShapeDtypeStruct((SAMP_N, 128), jnp.float32),
      ],
      mesh=mesh,
      scratch_types=[
          pltpu.VMEM((SEQ_PW,), jnp.int32),
          pltpu.VMEM((SAMP_PW,), jnp.int32),
          pltpu.VMEM((GCHUNK, 128), jnp.float32),
          pltpu.VMEM((GCHUNK, 128), jnp.float32),
          pltpu.SemaphoreType.DMA,
          pltpu.SemaphoreType.DMA,
      ],
  )
  return f(table_packed, seq_pidx, samp_pidx)


_BB = 16
_XRB = _BB * L
_NRB = _BB * K


def _extract32(x128, grp):
  out = jnp.zeros((x128.shape[0], D), jnp.float32)
  for k in range(PACK):
    piece = x128[:, k * D : (k + 1) * D]
    out = out + jnp.where(grp == k, piece, 0.0)
  return out


def _dense_body(xs_ref, g_ref, ns_ref, sg_ref, wself_ref, wn_ref, bs_ref,
                bl_ref, out_ref):
  ns32 = _extract32(ns_ref[...], sg_ref[...])
  rq = lax.broadcasted_iota(jnp.int32, (_BB, _NRB), 1) // K
  jq = lax.broadcasted_iota(jnp.int32, (_BB, _NRB), 0)
  q = (rq == jq).astype(jnp.float32)
  srow = jnp.dot(q, ns32, preferred_element_type=jnp.float32)
  rowb = jnp.dot(srow, wn_ref[...], preferred_element_type=jnp.float32)
  rowb = rowb + bl_ref[...]

  rp = lax.broadcasted_iota(jnp.int32, (_XRB, _BB), 0) // L
  jp = lax.broadcasted_iota(jnp.int32, (_XRB, _BB), 1)
  p = (rp == jp).astype(jnp.float32)
  rowb_exp = jnp.dot(p, rowb, preferred_element_type=jnp.float32)

  xs32 = _extract32(xs_ref[...], g_ref[...])
  y = jnp.dot(xs32, wself_ref[...], preferred_element_type=jnp.float32)
  y = y + bs_ref[...] + rowb_exp
  out_ref[...] = jnp.maximum(y, 0.0)


def _run_dense(xs, g, ns, sg, wself, wn, bs, bl):
  grid = B // _BB
  return pl.pallas_call(
      _dense_body,
      grid=(grid,),
      in_specs=[
          pl.BlockSpec((_XRB, 128), lambda i: (i, 0)),
          pl.BlockSpec((_XRB, 1), lambda i: (i, 0)),
          pl.BlockSpec((_NRB, 128), lambda i: (i, 0)),
          pl.BlockSpec((_NRB, 1), lambda i: (i, 0)),
          pl.BlockSpec((D, D), lambda i: (0, 0)),
          pl.BlockSpec((D, D), lambda i: (0, 0)),
          pl.BlockSpec((1, D), lambda i: (0, 0)),
          pl.BlockSpec((1, D), lambda i: (0, 0)),
      ],
      out_specs=pl.BlockSpec((_XRB, D), lambda i: (i, 0)),
      out_shape=jax.ShapeDtypeStruct((SEQ_N, D), jnp.float32),
  )(xs, g, ns, sg, wself, wn, bs, bl)


@jax.jit
def _kernel_impl(seq, mask, emb_item, linear_w, linear_b, self_linear_w,
                 self_linear_b):
  gum = jax.random.gumbel(jax.random.key(42), (B, L), dtype=jnp.float32)

  sampled = _run_topk(seq, mask, gum)

  table_packed = emb_item.reshape(VP, 128)
  seq_flat = seq.reshape(SEQ_N)
  samp_flat = sampled.reshape(SAMP_N)

  seq_rows, samp_rows = _run_sc_gather(
      table_packed, seq_flat >> 2, samp_flat >> 2)

  out = _run_dense(
      seq_rows, (seq_flat & 3).reshape(SEQ_N, 1),
      samp_rows, (samp_flat & 3).reshape(SAMP_N, 1),
      self_linear_w.T, linear_w.T / K,
      self_linear_b.reshape(1, D), linear_b.reshape(1, D))
  return out.reshape(B, L, D)


def kernel(user_indices, item_indices, seq, mask, emb_item, linear_w,
           linear_b, self_linear_w, self_linear_b):
  del user_indices, item_indices
  return _kernel_impl(seq, mask, emb_item, linear_w, linear_b,
                      self_linear_w, self_linear_b)


# speedup vs baseline: 12.8386x; 1.3115x over previous
import jax
import jax.numpy as jnp
from jax import lax
from jax.experimental import pallas as pl
from jax.experimental.pallas import tpu as pltpu
from jax.experimental.pallas import tpu_sc as plsc

B, L, K, V, D = 4096, 200, 20, 1000000, 32

NC, NS = 2, 16
NW = NC * NS

PACK = 128 // D
VP = V // PACK


_TOPK_ROWS = 256


def _topk_body(seq_ref, mask_ref, g_ref, out_ref):
  seq = seq_ref[...]
  mask = mask_ref[...]
  g = g_ref[...]
  neg_inf = jnp.float32(-jnp.inf)

  n_valid = jnp.sum(mask, axis=1, keepdims=True)
  w = jnp.where(n_valid > K, seq * mask, seq).astype(jnp.float32)
  logw = jnp.where(w > 0, jnp.log(jnp.maximum(w, 1e-30)), neg_inf)
  scores = jnp.where(logw > neg_inf, logw + g, neg_inf)

  col = lax.broadcasted_iota(jnp.int32, scores.shape, 1)
  big = jnp.int32(1 << 30)
  taken = jnp.zeros(scores.shape, jnp.bool_)
  for k in range(K):
    m = jnp.max(scores, axis=1, keepdims=True)
    is_max = (scores == m) & (~taken)
    idx = jnp.min(jnp.where(is_max, col, big), axis=1, keepdims=True)
    sel = col == idx
    val = jnp.sum(jnp.where(sel, seq, 0), axis=1, keepdims=True)
    out_ref[:, k : k + 1] = val
    taken = taken | sel
    scores = jnp.where(sel, neg_inf, scores)


def _run_topk(seq, mask, g):
  grid = B // _TOPK_ROWS
  return pl.pallas_call(
      _topk_body,
      grid=(grid,),
      in_specs=[
          pl.BlockSpec((_TOPK_ROWS, L), lambda i: (i, 0)),
          pl.BlockSpec((_TOPK_ROWS, L), lambda i: (i, 0)),
          pl.BlockSpec((_TOPK_ROWS, L), lambda i: (i, 0)),
      ],
      out_specs=pl.BlockSpec((_TOPK_ROWS, K), lambda i: (i, 0)),
      out_shape=jax.ShapeDtypeStruct((B, K), jnp.int32),
  )(seq, mask, g)


SEQ_N = B * L
SAMP_N = B * K
GCHUNK = 256


def _sc_gather_body(n_rows, table, idx_hbm, out, idx_v, rows_a, rows_b,
                    sem_a, sem_b):
  wid = lax.axis_index("s") * NC + lax.axis_index("c")
  per_w = n_rows // NW
  n_chunks = per_w // GCHUNK
  base = wid * per_w

  pltpu.sync_copy(idx_hbm.at[pl.ds(base, per_w)], idx_v)

  def start(j, buf, sem):
    pltpu.async_copy(
        table.at[idx_v.at[pl.ds(j * GCHUNK, GCHUNK)]], buf, sem)

  def drain(j, buf, sem):
    pltpu.make_async_copy(
        table.at[idx_v.at[pl.ds(j * GCHUNK, GCHUNK)]], buf, sem).wait()
    pltpu.sync_copy(buf, out.at[pl.ds(base + j * GCHUNK, GCHUNK)])

  start(0, rows_a, sem_a)

  def pair(t, _):
    j = t * 2
    start(j + 1, rows_b, sem_b)
    drain(j, rows_a, sem_a)
    start(j + 2, rows_a, sem_a)
    drain(j + 1, rows_b, sem_b)
    return 0

  lax.fori_loop(0, n_chunks // 2 - 1, pair, 0)
  j = n_chunks - 2
  start(j + 1, rows_b, sem_b)
  drain(j, rows_a, sem_a)
  drain(j + 1, rows_b, sem_b)


def _make_sc_gather(n_rows):
  mesh = plsc.VectorSubcoreMesh(
      core_axis_name="c", subcore_axis_name="s", num_cores=NC,
      num_subcores=NS)

  def body(table, idx_hbm, out, idx_v, rows_a, rows_b, sem_a, sem_b):
    _sc_gather_body(n_rows, table, idx_hbm, out, idx_v, rows_a, rows_b,
                    sem_a, sem_b)

  return pl.kernel(
      body,
      out_type=jax.ShapeDtypeStruct((n_rows, 128), jnp.float32),
      mesh=mesh,
      scratch_types=[
          pltpu.VMEM((n_rows // NW,), jnp.int32),
          pltpu.VMEM((GCHUNK, 128), jnp.float32),
          pltpu.VMEM((GCHUNK, 128), jnp.float32),
          pltpu.SemaphoreType.DMA,
          pltpu.SemaphoreType.DMA,
      ],
  )


_sc_gather_seq = _make_sc_gather(SEQ_N)
_sc_gather_samp = _make_sc_gather(SAMP_N)


_BB = 16
_XRB = _BB * L
_NRB = _BB * K


def _dense_body(xs_ref, g_ref, ns_ref, sg_ref, q_ref, p_ref, wsx4_ref,
                wnx4_ref, bs_ref, bl_ref, out_ref):
  lane_n = lax.broadcasted_iota(jnp.int32, (_NRB, 128), 1) // D
  nm = jnp.where(lane_n == sg_ref[...], ns_ref[...], 0.0)
  srow = jnp.dot(q_ref[...], nm, preferred_element_type=jnp.float32)
  rowb = jnp.dot(srow, wnx4_ref[...], preferred_element_type=jnp.float32)
  rowb = rowb + bl_ref[...]
  rowb_exp = jnp.dot(p_ref[...], rowb, preferred_element_type=jnp.float32)

  lane_x = lax.broadcasted_iota(jnp.int32, (_XRB, 128), 1) // D
  xm = jnp.where(lane_x == g_ref[...], xs_ref[...], 0.0)
  y = jnp.dot(xm, wsx4_ref[...], preferred_element_type=jnp.float32)
  out_ref[...] = jnp.maximum(y + bs_ref[...] + rowb_exp, 0.0)


def _run_dense(xs, g, ns, sg, q, p, wsx4, wnx4, bs, bl):
  grid = B // _BB
  return pl.pallas_call(
      _dense_body,
      grid=(grid,),
      in_specs=[
          pl.BlockSpec((_XRB, 128), lambda i: (i, 0)),
          pl.BlockSpec((_XRB, 1), lambda i: (i, 0)),
          pl.BlockSpec((_NRB, 128), lambda i: (i, 0)),
          pl.BlockSpec((_NRB, 1), lambda i: (i, 0)),
          pl.BlockSpec((_BB, _NRB), lambda i: (0, 0)),
          pl.BlockSpec((_XRB, _BB), lambda i: (0, 0)),
          pl.BlockSpec((128, D), lambda i: (0, 0)),
          pl.BlockSpec((128, D), lambda i: (0, 0)),
          pl.BlockSpec((1, D), lambda i: (0, 0)),
          pl.BlockSpec((1, D), lambda i: (0, 0)),
      ],
      out_specs=pl.BlockSpec((_XRB, D), lambda i: (i, 0)),
      out_shape=jax.ShapeDtypeStruct((SEQ_N, D), jnp.float32),
  )(xs, g, ns, sg, q, p, wsx4, wnx4, bs, bl)


@jax.jit
def _kernel_impl(seq, mask, emb_item, linear_w, linear_b, self_linear_w,
                 self_linear_b):
  gum = jax.random.gumbel(jax.random.key(42), (B, L), dtype=jnp.float32)

  table_packed = emb_item.reshape(VP, 128)
  seq_flat = seq.reshape(SEQ_N)

  seq_rows = _sc_gather_seq(table_packed, seq_flat >> 2)

  sampled = _run_topk(seq, mask, gum)
  samp_flat = sampled.reshape(SAMP_N)
  samp_rows = _sc_gather_samp(table_packed, samp_flat >> 2)

  q = jnp.repeat(jnp.eye(_BB, dtype=jnp.float32), K, axis=1)
  p = jnp.repeat(jnp.eye(_BB, dtype=jnp.float32), L, axis=0)
  wsx4 = jnp.tile(self_linear_w.T, (PACK, 1))
  wnx4 = jnp.tile(linear_w.T / K, (PACK, 1))

  out = _run_dense(
      seq_rows, (seq_flat & 3).reshape(SEQ_N, 1),
      samp_rows, (samp_flat & 3).reshape(SAMP_N, 1),
      q, p, wsx4, wnx4,
      self_linear_b.reshape(1, D), linear_b.reshape(1, D))
  return out.reshape(B, L, D)


def kernel(user_indices, item_indices, seq, mask, emb_item, linear_w,
           linear_b, self_linear_w, self_linear_b):
  del user_indices, item_indices
  return _kernel_impl(seq, mask, emb_item, linear_w, linear_b,
                      self_linear_w, self_linear_b)


# speedup vs baseline: 13.8159x; 1.0761x over previous
import jax
import jax.numpy as jnp
from jax import lax
from jax.experimental import pallas as pl
from jax.experimental.pallas import tpu as pltpu
from jax.experimental.pallas import tpu_sc as plsc

B, L, K, V, D = 4096, 200, 20, 1000000, 32

NC, NS = 2, 16
NW = NC * NS

PACK = 128 // D
VP = V // PACK


_TOPK_ROWS = 256


def _topk_body(seq_ref, mask_ref, g_ref, out_ref):
  seq = seq_ref[...]
  mask = mask_ref[...]
  g = g_ref[...]
  neg_inf = jnp.float32(-jnp.inf)

  n_valid = jnp.sum(mask, axis=1, keepdims=True)
  w = jnp.where(n_valid > K, seq * mask, seq).astype(jnp.float32)
  logw = jnp.where(w > 0, jnp.log(jnp.maximum(w, 1e-30)), neg_inf)
  scores = jnp.where(logw > neg_inf, logw + g, neg_inf)

  col = lax.broadcasted_iota(jnp.int32, scores.shape, 1)
  big = jnp.int32(1 << 30)
  taken = jnp.zeros(scores.shape, jnp.bool_)
  for k in range(K):
    m = jnp.max(scores, axis=1, keepdims=True)
    is_max = (scores == m) & (~taken)
    idx = jnp.min(jnp.where(is_max, col, big), axis=1, keepdims=True)
    sel = col == idx
    val = jnp.sum(jnp.where(sel, seq, 0), axis=1, keepdims=True)
    out_ref[:, k : k + 1] = val
    taken = taken | sel
    scores = jnp.where(sel, neg_inf, scores)


def _run_topk(seq, mask, g):
  grid = B // _TOPK_ROWS
  return pl.pallas_call(
      _topk_body,
      grid=(grid,),
      in_specs=[
          pl.BlockSpec((_TOPK_ROWS, L), lambda i: (i, 0)),
          pl.BlockSpec((_TOPK_ROWS, L), lambda i: (i, 0)),
          pl.BlockSpec((_TOPK_ROWS, L), lambda i: (i, 0)),
      ],
      out_specs=pl.BlockSpec((_TOPK_ROWS, K), lambda i: (i, 0)),
      out_shape=jax.ShapeDtypeStruct((B, K), jnp.int32),
  )(seq, mask, g)


SEQ_N = B * L
SAMP_N = B * K
GCHUNK = 256


def _sc_gather_body(n_rows, table, idx_hbm, out, idx_v, rows_a, rows_b,
                    sem_a, sem_b):
  wid = lax.axis_index("s") * NC + lax.axis_index("c")
  per_w = n_rows // NW
  n_chunks = per_w // GCHUNK
  base = wid * per_w

  pltpu.sync_copy(idx_hbm.at[pl.ds(base, per_w)], idx_v)

  def start(j, buf, sem):
    pltpu.async_copy(
        table.at[idx_v.at[pl.ds(j * GCHUNK, GCHUNK)]], buf, sem)

  def drain(j, buf, sem):
    pltpu.make_async_copy(
        table.at[idx_v.at[pl.ds(j * GCHUNK, GCHUNK)]], buf, sem).wait()
    pltpu.sync_copy(buf, out.at[pl.ds(base + j * GCHUNK, GCHUNK)])

  start(0, rows_a, sem_a)

  def pair(t, _):
    j = t * 2
    start(j + 1, rows_b, sem_b)
    drain(j, rows_a, sem_a)
    start(j + 2, rows_a, sem_a)
    drain(j + 1, rows_b, sem_b)
    return 0

  lax.fori_loop(0, n_chunks // 2 - 1, pair, 0)
  j = n_chunks - 2
  start(j + 1, rows_b, sem_b)
  drain(j, rows_a, sem_a)
  drain(j + 1, rows_b, sem_b)


def _make_sc_gather(n_rows):
  mesh = plsc.VectorSubcoreMesh(
      core_axis_name="c", subcore_axis_name="s", num_cores=NC,
      num_subcores=NS)

  def body(table, idx_hbm, out, idx_v, rows_a, rows_b, sem_a, sem_b):
    _sc_gather_body(n_rows, table, idx_hbm, out, idx_v, rows_a, rows_b,
                    sem_a, sem_b)

  return pl.kernel(
      body,
      out_type=jax.ShapeDtypeStruct((n_rows, 128), jnp.float32),
      mesh=mesh,
      scratch_types=[
          pltpu.VMEM((n_rows // NW,), jnp.int32),
          pltpu.VMEM((GCHUNK, 128), jnp.float32),
          pltpu.VMEM((GCHUNK, 128), jnp.float32),
          pltpu.SemaphoreType.DMA,
          pltpu.SemaphoreType.DMA,
      ],
  )


_sc_gather_seq = _make_sc_gather(SEQ_N)
_sc_gather_samp = _make_sc_gather(SAMP_N)


_BB = 16
_XRB = _BB * L
_NRB = _BB * K


def _dense_body(xs_ref, seq_ref, ns_ref, samp_ref, wsx4_ref, wnx4_ref,
                bs_ref, bl_ref, out_ref):
  lane_n = lax.broadcasted_iota(jnp.int32, (_BB, K, 128), 2) // D
  nm = jnp.where(lane_n == (samp_ref[...] & 3)[:, :, None], ns_ref[...], 0.0)
  srow = jnp.sum(nm, axis=1)
  rowb = jnp.dot(srow, wnx4_ref[...], preferred_element_type=jnp.float32)
  rowb = rowb + bl_ref[...]

  lane_x = lax.broadcasted_iota(jnp.int32, (_BB, L, 128), 2) // D
  xm = jnp.where(lane_x == (seq_ref[...] & 3)[:, :, None], xs_ref[...], 0.0)
  y = lax.dot_general(
      xm, wsx4_ref[...], (((2,), (0,)), ((), ())),
      preferred_element_type=jnp.float32)
  out_ref[...] = jnp.maximum(y + bs_ref[...] + rowb[:, None, :], 0.0)


def _run_dense(xs3, seq, ns3, sampled, wsx4, wnx4, bs, bl):
  grid = B // _BB
  return pl.pallas_call(
      _dense_body,
      grid=(grid,),
      in_specs=[
          pl.BlockSpec((_BB, L, 128), lambda i: (i, 0, 0)),
          pl.BlockSpec((_BB, L), lambda i: (i, 0)),
          pl.BlockSpec((_BB, K, 128), lambda i: (i, 0, 0)),
          pl.BlockSpec((_BB, K), lambda i: (i, 0)),
          pl.BlockSpec((128, D), lambda i: (0, 0)),
          pl.BlockSpec((128, D), lambda i: (0, 0)),
          pl.BlockSpec((1, D), lambda i: (0, 0)),
          pl.BlockSpec((1, D), lambda i: (0, 0)),
      ],
      out_specs=pl.BlockSpec((_BB, L, D), lambda i: (i, 0, 0)),
      out_shape=jax.ShapeDtypeStruct((B, L, D), jnp.float32),
  )(xs3, seq, ns3, sampled, wsx4, wnx4, bs, bl)


@jax.jit
def _kernel_impl(seq, mask, emb_item, linear_w, linear_b, self_linear_w,
                 self_linear_b):
  gum = jax.random.gumbel(jax.random.key(42), (B, L), dtype=jnp.float32)

  table_packed = emb_item.reshape(VP, 128)
  seq_flat = seq.reshape(SEQ_N)

  seq_rows = _sc_gather_seq(table_packed, seq_flat >> 2)

  sampled = _run_topk(seq, mask, gum)
  samp_rows = _sc_gather_samp(table_packed, sampled.reshape(SAMP_N) >> 2)

  wsx4 = jnp.tile(self_linear_w.T, (PACK, 1))
  wnx4 = jnp.tile(linear_w.T / K, (PACK, 1))

  return _run_dense(
      seq_rows.reshape(B, L, 128), seq,
      samp_rows.reshape(B, K, 128), sampled,
      wsx4, wnx4,
      self_linear_b.reshape(1, D), linear_b.reshape(1, D))


def kernel(user_indices, item_indices, seq, mask, emb_item, linear_w,
           linear_b, self_linear_w, self_linear_b):
  del user_indices, item_indices
  return _kernel_impl(seq, mask, emb_item, linear_w, linear_b,
                      self_linear_w, self_linear_b)
